# baseline (speedup 1.0000x reference)
import functools

import jax
import jax.numpy as jnp
from jax import lax
from jax.experimental import pallas as pl
from jax.experimental.pallas import tpu as pltpu
from jax.experimental.pallas import tpu_sc as plsc

C0_END = 20000
C1_END = 200000
D_PROJ = 128
SCALE = float(D_PROJ) ** 0.5

T = 4096 * 50
NW = 32
ROWS_W = 4096 // NW
RPC = 2
NCHUNK = ROWS_W // RPC
SLOTS = RPC * 128
CTOK = RPC * 50
TOK_TILE = 800

_ZV = None


def _sc_gather_body(inp_hbm, w0_hbm, emb1_hbm, emb2_hbm, g_hbm, h_hbm,
                    tok_v, idx0_v, idx1_v, idx2_v,
                    rows0_v, rows1_v, rows2_v, g_v, h_v,
                    sem0, sem1, sem2):
    wid = lax.axis_index("s") * 2 + lax.axis_index("c")
    row_base = wid * ROWS_W
    IOTA16 = lax.iota(jnp.int32, 16)
    ZERO16 = jnp.zeros((16,), jnp.float32)

    def chunk_body(k, carry):
        row0 = row_base + k * RPC
        off = row0 * 50
        pltpu.sync_copy(inp_hbm.at[pl.ds(row0 * 128, SLOTS)], tok_v)
        for g in range(SLOTS // 16):
            sl = pl.ds(g * 16, 16)
            t = tok_v[sl]
            real = ((IOTA16 + 16 * g) & 127) < 50
            m0 = real & (t < C0_END)
            m1 = real & (t >= C0_END) & (t < C1_END)
            m2 = real & (t >= C1_END)
            idx0_v[sl] = jnp.where(m0, t, -1)
            idx1_v[sl] = jnp.where(m1, t - C0_END, -1)
            idx2_v[sl] = jnp.where(m2, t - C1_END, -1)
        cps = []
        for half in range(RPC):
            hs = pl.ds(half * 128, 128)
            cps.append(pltpu.async_copy(
                w0_hbm.at[plsc.Indices(idx0_v.at[hs], ignored_value=-1)],
                rows0_v.at[hs], sem0))
            cps.append(pltpu.async_copy(
                emb1_hbm.at[plsc.Indices(idx1_v.at[hs], ignored_value=-1)],
                rows1_v.at[hs], sem1))
            cps.append(pltpu.async_copy(
                emb2_hbm.at[plsc.Indices(idx2_v.at[hs], ignored_value=-1)],
                rows2_v.at[pl.ds(half * 128, 128)], sem2))

        def zero_body(r, carry2):
            for q in range(8):
                g_v[r, pl.ds(16 * q, 16)] = ZERO16
            for q in range(3):
                h_v[r, pl.ds(16 * q, 16)] = ZERO16
            return carry2

        lax.fori_loop(0, CTOK, zero_body, 0)
        for cp in cps:
            cp.wait()

        def grp_body(gi, carry2):
            tvec = tok_v[pl.ds(16 * gi, 16)]
            base_slot = 16 * gi
            for j in range(16):
                tk = tvec[j]
                s = base_slot + j
                col = s & 127
                dest = (s >> 7) * 50 + col

                @pl.when((col < 50) & (tk < C0_END))
                def _():
                    for q in range(8):
                        g_v[dest, pl.ds(16 * q, 16)] = (
                            rows0_v[s, pl.ds(16 * q, 16)])

                @pl.when((col < 50) & (tk >= C0_END) & (tk < C1_END))
                def _():
                    for h in range(2):
                        h_v[dest, pl.ds(16 * h, 16)] = (
                            rows1_v[s, pl.ds(16 * h, 16)])

                @pl.when((col < 50) & (tk >= C1_END))
                def _():
                    rowv = s + (IOTA16 >> 3)
                    colv = IOTA16 & 7
                    pair = plsc.load_gather(rows2_v, [rowv, colv])
                    h_v[dest, pl.ds(32, 16)] = jnp.where(IOTA16 < 8, pair, 0.0)

            return carry2

        lax.fori_loop(0, SLOTS // 16, grp_body, 0)
        pltpu.sync_copy(g_v, g_hbm.at[pl.ds(off, CTOK)])
        pltpu.sync_copy(h_v, h_hbm.at[pl.ds(off, CTOK), pl.ds(0, 48)])
        return carry

    lax.fori_loop(0, NCHUNK, chunk_body, 0)


@functools.lru_cache(maxsize=1)
def _sc_gather():
    return pl.kernel(
        _sc_gather_body,
        mesh=plsc.VectorSubcoreMesh(core_axis_name="c", subcore_axis_name="s"),
        out_type=(jax.ShapeDtypeStruct((T, 128), jnp.float32),
                  jax.ShapeDtypeStruct((T, 128), jnp.float32)),
        scratch_types=[
            pltpu.VMEM((SLOTS,), jnp.int32),
            pltpu.VMEM((SLOTS,), jnp.int32),
            pltpu.VMEM((SLOTS,), jnp.int32),
            pltpu.VMEM((SLOTS,), jnp.int32),
            pltpu.VMEM((SLOTS, 128), jnp.float32),
            pltpu.VMEM((SLOTS, 32), jnp.float32),
            pltpu.VMEM((SLOTS + 2, 8), jnp.float32),
            pltpu.VMEM((CTOK, 128), jnp.float32),
            pltpu.VMEM((CTOK, 48), jnp.float32),
            pltpu.SemaphoreType.DMA,
            pltpu.SemaphoreType.DMA,
            pltpu.SemaphoreType.DMA,
        ],
        compiler_params=pltpu.CompilerParams(
            use_tc_tiling_on_sc=False, needs_layout_passes=False),
    )


def _w0_body(e_ref, p_ref, o_ref):
    o_ref[...] = lax.dot_general(
        e_ref[...], p_ref[...], (((1,), (1,)), ((), ())),
        preferred_element_type=jnp.float32)


def _w0(emb0, proj0):
    return pl.pallas_call(
        _w0_body,
        grid=(10,),
        in_specs=[
            pl.BlockSpec((2000, 128), lambda i: (i, 0)),
            pl.BlockSpec((128, 128), lambda i: (0, 0)),
        ],
        out_specs=pl.BlockSpec((2000, 128), lambda i: (i, 0)),
        out_shape=jax.ShapeDtypeStruct((20000, 128), jnp.float32),
    )(emb0, proj0)


def _tc_project_body(g_ref, h_ref, q_ref, o_ref):
    y = g_ref[...] + lax.dot_general(
        h_ref[:, :48], q_ref[...], (((1,), (0,)), ((), ())),
        preferred_element_type=jnp.float32)
    y = y * SCALE
    o_ref[...] = y.reshape(o_ref.shape)


def _tc_project(g, h, q):
    rows = 4096 // (T // TOK_TILE)
    return pl.pallas_call(
        _tc_project_body,
        grid=(T // TOK_TILE,),
        in_specs=[
            pl.BlockSpec((TOK_TILE, 128), lambda i: (i, 0)),
            pl.BlockSpec((TOK_TILE, 128), lambda i: (i, 0)),
            pl.BlockSpec((48, 128), lambda i: (0, 0)),
        ],
        out_specs=pl.BlockSpec((rows, 50, 128), lambda i: (i, 0, 0)),
        out_shape=jax.ShapeDtypeStruct((4096, 50, 128), jnp.float32),
    )(g, h, q)


def kernel(inp, emb0, emb1, emb2, proj0, proj1, proj2):
    inp_pad = jnp.pad(inp, ((0, 0), (0, 78))).reshape(-1)
    w0 = _w0(emb0, proj0)
    g, h = _sc_gather()(inp_pad, w0, emb1, emb2)
    q = jnp.zeros((48, 128), jnp.float32)
    q = q.at[0:32, :].set(proj1.T).at[32:40, :].set(proj2.T)
    return _tc_project(g, h, q)


# speedup vs baseline: 1.4437x; 1.4437x over previous
import functools

import jax
import jax.numpy as jnp
from jax import lax
from jax.experimental import pallas as pl
from jax.experimental.pallas import tpu as pltpu
from jax.experimental.pallas import tpu_sc as plsc

C0_END = 20000
C1_END = 200000
D_PROJ = 128
SCALE = float(D_PROJ) ** 0.5

T = 4096 * 50
NW = 32
TW = T // NW
CHUNK = 128
NCHUNK = TW // CHUNK
TOK_TILE = 800

_ZV = None


def _sc_gather_body(inp_hbm, w0_hbm, emb1_hbm, emb2_hbm, g_hbm, h_hbm,
                    tok_v, idx0_v, idx1_v, idx2_v,
                    rows0_v, rows1_v, rows2_v, g_v, h_v,
                    sem0, sem1, sem2):
    wid = lax.axis_index("s") * 2 + lax.axis_index("c")
    base = wid * TW
    IOTA16 = lax.iota(jnp.int32, 16)
    ZERO16 = jnp.zeros((16,), jnp.float32)

    def chunk_body(k, carry):
        off = base + k * CHUNK
        pltpu.sync_copy(inp_hbm.at[pl.ds(off, CHUNK)], tok_v)
        for g in range(CHUNK // 16):
            sl = pl.ds(g * 16, 16)
            t = tok_v[sl]
            m0 = t < C0_END
            m1 = (t >= C0_END) & (t < C1_END)
            m2 = t >= C1_END
            idx0_v[sl] = jnp.where(m0, t, -1)
            idx1_v[sl] = jnp.where(m1, t - C0_END, -1)
            idx2_v[sl] = jnp.where(m2, t - C1_END, -1)
        cp0 = pltpu.async_copy(
            w0_hbm.at[plsc.Indices(idx0_v, ignored_value=-1)], rows0_v, sem0)
        cp1 = pltpu.async_copy(
            emb1_hbm.at[plsc.Indices(idx1_v, ignored_value=-1)], rows1_v, sem1)
        cp2 = pltpu.async_copy(
            emb2_hbm.at[plsc.Indices(idx2_v, ignored_value=-1)],
            rows2_v.at[pl.ds(0, CHUNK)], sem2)

        def zero_body(r, carry2):
            for q in range(8):
                g_v[r, pl.ds(16 * q, 16)] = ZERO16
            for q in range(3):
                h_v[r, pl.ds(16 * q, 16)] = ZERO16
            return carry2

        lax.fori_loop(0, CHUNK, zero_body, 0)
        cp0.wait()
        cp1.wait()
        cp2.wait()

        def grp_body(gi, carry2):
            tvec = tok_v[pl.ds(16 * gi, 16)]
            for j in range(16):
                tk = tvec[j]
                t = 16 * gi + j

                @pl.when(tk < C0_END)
                def _():
                    for q in range(8):
                        g_v[t, pl.ds(16 * q, 16)] = rows0_v[t, pl.ds(16 * q, 16)]

                @pl.when((tk >= C0_END) & (tk < C1_END))
                def _():
                    for h in range(2):
                        h_v[t, pl.ds(16 * h, 16)] = rows1_v[t, pl.ds(16 * h, 16)]

                @pl.when(tk >= C1_END)
                def _():
                    rowv = t + (IOTA16 >> 3)
                    colv = IOTA16 & 7
                    pair = plsc.load_gather(rows2_v, [rowv, colv])
                    h_v[t, pl.ds(32, 16)] = jnp.where(IOTA16 < 8, pair, 0.0)

            return carry2

        lax.fori_loop(0, CHUNK // 16, grp_body, 0)
        pltpu.sync_copy(g_v, g_hbm.at[pl.ds(off, CHUNK)])
        pltpu.sync_copy(h_v, h_hbm.at[pl.ds(off, CHUNK), pl.ds(0, 48)])
        return carry

    lax.fori_loop(0, NCHUNK, chunk_body, 0)


@functools.lru_cache(maxsize=1)
def _sc_gather():
    return pl.kernel(
        _sc_gather_body,
        mesh=plsc.VectorSubcoreMesh(core_axis_name="c", subcore_axis_name="s"),
        out_type=(jax.ShapeDtypeStruct((T, 128), jnp.float32),
                  jax.ShapeDtypeStruct((T, 128), jnp.float32)),
        scratch_types=[
            pltpu.VMEM((CHUNK,), jnp.int32),
            pltpu.VMEM((CHUNK,), jnp.int32),
            pltpu.VMEM((CHUNK,), jnp.int32),
            pltpu.VMEM((CHUNK,), jnp.int32),
            pltpu.VMEM((CHUNK, 128), jnp.float32),
            pltpu.VMEM((CHUNK, 32), jnp.float32),
            pltpu.VMEM((CHUNK + 2, 8), jnp.float32),
            pltpu.VMEM((CHUNK, 128), jnp.float32),
            pltpu.VMEM((CHUNK, 48), jnp.float32),
            pltpu.SemaphoreType.DMA,
            pltpu.SemaphoreType.DMA,
            pltpu.SemaphoreType.DMA,
        ],
        compiler_params=pltpu.CompilerParams(
            use_tc_tiling_on_sc=False, needs_layout_passes=False),
    )


def _w0_body(e_ref, p_ref, o_ref):
    o_ref[...] = lax.dot_general(
        e_ref[...], p_ref[...], (((1,), (1,)), ((), ())),
        preferred_element_type=jnp.float32)


def _w0(emb0, proj0):
    return pl.pallas_call(
        _w0_body,
        grid=(10,),
        in_specs=[
            pl.BlockSpec((2000, 128), lambda i: (i, 0)),
            pl.BlockSpec((128, 128), lambda i: (0, 0)),
        ],
        out_specs=pl.BlockSpec((2000, 128), lambda i: (i, 0)),
        out_shape=jax.ShapeDtypeStruct((20000, 128), jnp.float32),
    )(emb0, proj0)


def _tc_project_body(g_ref, h_ref, q_ref, o_ref):
    y = g_ref[...] + lax.dot_general(
        h_ref[:, :48], q_ref[...], (((1,), (0,)), ((), ())),
        preferred_element_type=jnp.float32)
    y = y * SCALE
    o_ref[...] = y.reshape(o_ref.shape)


def _tc_project(g, h, q):
    rows = 4096 // (T // TOK_TILE)
    return pl.pallas_call(
        _tc_project_body,
        grid=(T // TOK_TILE,),
        in_specs=[
            pl.BlockSpec((TOK_TILE, 128), lambda i: (i, 0)),
            pl.BlockSpec((TOK_TILE, 128), lambda i: (i, 0)),
            pl.BlockSpec((48, 128), lambda i: (0, 0)),
        ],
        out_specs=pl.BlockSpec((rows, 50, 128), lambda i: (i, 0, 0)),
        out_shape=jax.ShapeDtypeStruct((4096, 50, 128), jnp.float32),
    )(g, h, q)


def kernel(inp, emb0, emb1, emb2, proj0, proj1, proj2):
    inp_flat = inp.reshape(-1)
    w0 = _w0(emb0, proj0)
    g, h = _sc_gather()(inp_flat, w0, emb1, emb2)
    q = jnp.zeros((48, 128), jnp.float32)
    q = q.at[0:32, :].set(proj1.T).at[32:40, :].set(proj2.T)
    return _tc_project(g, h, q)
